# baseline (speedup 1.0000x reference)
import functools

import jax
import jax.numpy as jnp
import numpy as np
from jax import lax
from jax.experimental import pallas as pl
from jax.experimental.pallas import tpu as pltpu
from jax.experimental.pallas import tpu_sc as plsc

NC, NS, L = 2, 16, 16
NW = NC * NS
B, S, E, C = 4096, 200, 64, 2
BPW = B // NW
ROWS_PW = BPW * S
CHUNK = 128
NCHUNK = ROWS_PW // CHUNK
ACC_ROWS = NS * BPW
NPAD = 128

_SEG = (np.arange(NS)[:, None] * BPW
        + np.repeat(np.arange(BPW), S)[None, :]).astype(np.int32)
_SEG = _SEG.reshape(NS, NCHUNK, CHUNK)

_mesh = plsc.VectorSubcoreMesh(core_axis_name="c", subcore_axis_name="s")


@functools.partial(
    pl.kernel,
    out_type=jax.ShapeDtypeStruct((B, E), jnp.float32),
    mesh=_mesh,
    compiler_params=pltpu.CompilerParams(use_tc_tiling_on_sc=False),
    scratch_types=[
        pltpu.VMEM((NCHUNK, CHUNK), jnp.int32),
        pltpu.VMEM((NCHUNK, CHUNK), jnp.int32),
        pltpu.VMEM((2, CHUNK, E), jnp.float32),
        pltpu.VMEM_SHARED((ACC_ROWS, E), jnp.float32),
        pltpu.SemaphoreType.DMA,
    ],
)
def _sc_pool(table, xr, seg, zeros, out, idx_v, seg_v, rows_v, acc_sh, sem):
    c = lax.axis_index("c")
    s = lax.axis_index("s")

    pltpu.sync_copy(zeros, acc_sh.at[pl.ds(s * BPW, BPW)])
    pltpu.sync_copy(xr.at[c, s], idx_v)
    pltpu.sync_copy(seg.at[s], seg_v)
    plsc.subcore_barrier()

    def start_gather(k, slot):
        return pltpu.async_copy(table.at[idx_v.at[k]], rows_v.at[slot], sem)

    def wait_gather(k, slot):
        pltpu.make_async_copy(table.at[idx_v.at[k]], rows_v.at[slot], sem
                              ).wait()

    start_gather(0, 0)

    def chunk_body(k, carry):
        slot = lax.rem(k, 2)
        start_gather(k + 1, 1 - slot)
        wait_gather(k, slot)
        pltpu.sync_copy(rows_v.at[slot], acc_sh.at[seg_v.at[k]], add=True)
        return carry

    lax.fori_loop(0, NCHUNK - 1, chunk_body, 0)
    last = NCHUNK - 1
    wait_gather(last, last % 2)
    pltpu.sync_copy(rows_v.at[last % 2], acc_sh.at[seg_v.at[last]], add=True)
    plsc.subcore_barrier()

    pltpu.sync_copy(acc_sh.at[pl.ds(s * BPW, BPW)],
                    out.at[pl.ds((c * NS + s) * BPW, BPW)])


def _tc_head_body(pooled_ref, w_ref, b_ref, out_ref):
    out_ref[...] = (
        jnp.dot(pooled_ref[...], w_ref[...],
                preferred_element_type=jnp.float32)
        + b_ref[...]
    )


_tc_head = pl.pallas_call(
    _tc_head_body,
    out_shape=jax.ShapeDtypeStruct((B, NPAD), jnp.float32),
)


def kernel(x, embed_table, fc_w, fc_b):
    xr = x.reshape(NC, NS, NCHUNK, CHUNK).astype(jnp.int32)
    zeros = jnp.zeros((BPW, E), jnp.float32)
    pooled = _sc_pool(embed_table, xr, jnp.asarray(_SEG), zeros)
    w = jnp.zeros((E, NPAD), jnp.float32).at[:, :C].set(fc_w.T * (1.0 / S))
    bias = jnp.zeros((1, NPAD), jnp.float32).at[0, :C].set(fc_b)
    return _tc_head(pooled, w, bias)[:, :C]


# speedup vs baseline: 3.4139x; 3.4139x over previous
import functools

import jax
import jax.numpy as jnp
import numpy as np
from jax import lax
from jax.experimental import pallas as pl
from jax.experimental.pallas import tpu as pltpu
from jax.experimental.pallas import tpu_sc as plsc

NC, NS, L = 2, 16, 16
NW = NC * NS
V, B, S, E, C = 1000000, 4096, 200, 64, 2
BPW = B // NW
RING = 8
NBLK = 8192
GRID = -(-V // NBLK)

_mesh = plsc.VectorSubcoreMesh(core_axis_name="c", subcore_axis_name="s")


def _project_body(w_ref, b_ref, tt_ref, o0_ref, o1_ref):
    m = jnp.dot(w_ref[...], tt_ref[...], preferred_element_type=jnp.float32)
    m = m + b_ref[...][:, 0:1]
    o0_ref[...] = m[0]
    o1_ref[...] = m[1]


_tc_project = pl.pallas_call(
    _project_body,
    grid=(GRID,),
    in_specs=[
        pl.BlockSpec((8, E), lambda i: (0, 0)),
        pl.BlockSpec((8, 128), lambda i: (0, 0)),
        pl.BlockSpec((E, NBLK), lambda i: (0, i)),
    ],
    out_specs=[
        pl.BlockSpec((NBLK,), lambda i: (i,)),
        pl.BlockSpec((NBLK,), lambda i: (i,)),
    ],
    out_shape=[
        jax.ShapeDtypeStruct((V,), jnp.float32),
        jax.ShapeDtypeStruct((V,), jnp.float32),
    ],
)


@functools.partial(
    pl.kernel,
    out_type=jax.ShapeDtypeStruct((C, B), jnp.float32),
    mesh=_mesh,
    compiler_params=pltpu.CompilerParams(use_tc_tiling_on_sc=False),
    scratch_types=[
        pltpu.VMEM((S, BPW), jnp.int32),
        pltpu.VMEM((RING, C, BPW), jnp.float32),
        pltpu.VMEM((C, BPW), jnp.float32),
        pltpu.SemaphoreType.DMA,
    ],
)
def _sc_pool(p0, p1, xr, out, idx_v, gbuf, av, sem):
    c = lax.axis_index("c")
    s = lax.axis_index("s")
    ps = (p0, p1)

    pltpu.sync_copy(xr.at[c, s], idx_v)

    def start(k, slot):
        for cls in range(C):
            pltpu.async_copy(ps[cls].at[idx_v.at[k]], gbuf.at[slot, cls], sem)

    def wait(k, slot):
        for cls in range(C):
            pltpu.make_async_copy(ps[cls].at[idx_v.at[k]],
                                  gbuf.at[slot, cls], sem).wait()

    for k in range(RING):
        start(k, k)

    def tok_body(k, carry):
        slot = lax.rem(k, RING)
        wait(k, slot)
        new = []
        for cls in range(C):
            for g in range(BPW // L):
                new.append(carry[cls * (BPW // L) + g]
                           + gbuf[slot, cls, pl.ds(g * L, L)])
        @pl.when(k + RING < S)
        def _():
            start(k + RING, slot)
        return tuple(new)

    zero = jnp.zeros((L,), jnp.float32)
    acc = lax.fori_loop(0, S, tok_body, (zero,) * (C * (BPW // L)))

    for cls in range(C):
        for g in range(BPW // L):
            av[cls, pl.ds(g * L, L)] = acc[cls * (BPW // L) + g]
    pltpu.sync_copy(av, out.at[:, pl.ds((c * NS + s) * BPW, BPW)])


def kernel(x, embed_table, fc_w, fc_b):
    tt = embed_table.T
    w8 = jnp.zeros((8, E), jnp.float32).at[:C].set(fc_w * (1.0 / S))
    b8 = jnp.zeros((8, 128), jnp.float32).at[:C, 0].set(fc_b * (1.0 / S))
    p0, p1 = _tc_project(w8, b8, tt)
    xr = (x.reshape(NC, NS, BPW, S).astype(jnp.int32)
          .transpose(0, 1, 3, 2))
    out = _sc_pool(p0, p1, xr)
    return out.T


# speedup vs baseline: 3.5933x; 1.0525x over previous
import functools

import jax
import jax.numpy as jnp
import numpy as np
from jax import lax
from jax.experimental import pallas as pl
from jax.experimental.pallas import tpu as pltpu
from jax.experimental.pallas import tpu_sc as plsc

NC, NS, L = 2, 16, 16
NW = NC * NS
V, B, S, E, C = 1000000, 4096, 200, 64, 2
BPW = B // NW
GPB = BPW // L
TPC = 4
CHUNK = TPC * BPW
RING = 5
NCHUNK = S // TPC
NBLK = 8192
GRID = -(-V // NBLK)

_mesh = plsc.VectorSubcoreMesh(core_axis_name="c", subcore_axis_name="s")


def _project_body(w_ref, b_ref, tt_ref, o0_ref, o1_ref):
    m = jnp.dot(w_ref[...], tt_ref[...], preferred_element_type=jnp.float32)
    m = m + b_ref[...][:, 0:1]
    o0_ref[...] = m[0]
    o1_ref[...] = m[1]


_tc_project = pl.pallas_call(
    _project_body,
    grid=(GRID,),
    in_specs=[
        pl.BlockSpec((8, E), lambda i: (0, 0)),
        pl.BlockSpec((8, 128), lambda i: (0, 0)),
        pl.BlockSpec((E, NBLK), lambda i: (0, i)),
    ],
    out_specs=[
        pl.BlockSpec((NBLK,), lambda i: (i,)),
        pl.BlockSpec((NBLK,), lambda i: (i,)),
    ],
    out_shape=[
        jax.ShapeDtypeStruct((V,), jnp.float32),
        jax.ShapeDtypeStruct((V,), jnp.float32),
    ],
)


@functools.partial(
    pl.kernel,
    out_type=jax.ShapeDtypeStruct((C, B), jnp.float32),
    mesh=_mesh,
    compiler_params=pltpu.CompilerParams(use_tc_tiling_on_sc=False),
    scratch_types=(
        [pltpu.VMEM((NCHUNK, CHUNK), jnp.int32)]
        + [pltpu.VMEM((CHUNK,), jnp.float32)
           for _ in range(C * RING)]
        + [pltpu.VMEM((C, BPW), jnp.float32),
           pltpu.SemaphoreType.DMA]
    ),
)
def _sc_pool(p0, p1, xr, out, idx_v, *rest):
    gbufs, av, sem = rest[:C * RING], rest[C * RING], rest[C * RING + 1]
    ps = (p0, p1)
    c = lax.axis_index("c")
    s = lax.axis_index("s")

    pltpu.sync_copy(xr.at[c, s], idx_v)

    def start(k, r):
        for cls in range(C):
            pltpu.async_copy(ps[cls].at[idx_v.at[k]],
                             gbufs[cls * RING + r], sem)

    def wait(k, r):
        for cls in range(C):
            pltpu.make_async_copy(ps[cls].at[idx_v.at[k]],
                                  gbufs[cls * RING + r], sem).wait()

    for k in range(RING):
        start(k, k)

    def grp_body(ko, carry):
        acc = list(carry)
        for r in range(RING):
            k = ko * RING + r
            wait(k, r)
            for cls in range(C):
                buf = gbufs[cls * RING + r]
                for t in range(TPC):
                    for g in range(GPB):
                        i = cls * GPB + g
                        acc[i] = acc[i] + buf[pl.ds(t * BPW + g * L, L)]
            @pl.when(ko + 1 < NCHUNK // RING)
            def _():
                start(k + RING, r)
        return tuple(acc)

    zero = jnp.zeros((L,), jnp.float32)
    acc = lax.fori_loop(0, NCHUNK // RING, grp_body, (zero,) * (C * GPB))

    for cls in range(C):
        for g in range(GPB):
            av[cls, pl.ds(g * L, L)] = acc[cls * GPB + g]
    pltpu.sync_copy(av, out.at[:, pl.ds((c * NS + s) * BPW, BPW)])


def kernel(x, embed_table, fc_w, fc_b):
    tt = embed_table.T
    w8 = jnp.zeros((8, E), jnp.float32).at[:C].set(fc_w * (1.0 / S))
    b8 = jnp.zeros((8, 128), jnp.float32).at[:C, 0].set(fc_b * (1.0 / S))
    p0, p1 = _tc_project(w8, b8, tt)
    xr = (x.reshape(NC, NS, BPW, S).astype(jnp.int32)
          .transpose(0, 1, 3, 2).reshape(NC, NS, NCHUNK, CHUNK))
    out = _sc_pool(p0, p1, xr)
    return out.T


# speedup vs baseline: 5.2433x; 1.4592x over previous
import functools

import jax
import jax.numpy as jnp
import numpy as np
from jax import lax
from jax.experimental import pallas as pl
from jax.experimental.pallas import tpu as pltpu
from jax.experimental.pallas import tpu_sc as plsc

NC, NS, L = 2, 16, 16
NW = NC * NS
V, B, S, E, C = 1000000, 4096, 200, 64, 2
BPW = B // NW
GPB = BPW // L
TPC = 4
CHUNK = TPC * BPW
RING = 10
NCHUNK = S // TPC
NBLK = 16384
GRID = -(-V // NBLK)

_mesh = plsc.VectorSubcoreMesh(core_axis_name="c", subcore_axis_name="s")


def _project_body(w_ref, b_ref, tt_ref, o_ref):
    m = jnp.dot(w_ref[...], tt_ref[...], preferred_element_type=jnp.float32)
    m = m + b_ref[...][:, 0:1]
    u0 = lax.bitcast_convert_type(m[0].astype(jnp.bfloat16), jnp.uint16)
    u1 = lax.bitcast_convert_type(m[1].astype(jnp.bfloat16), jnp.uint16)
    o_ref[...] = (u0.astype(jnp.uint32)
                  | (u1.astype(jnp.uint32) << jnp.uint32(16)))


_tc_project = pl.pallas_call(
    _project_body,
    grid=(GRID,),
    in_specs=[
        pl.BlockSpec((8, E), lambda i: (0, 0)),
        pl.BlockSpec((8, 128), lambda i: (0, 0)),
        pl.BlockSpec((E, NBLK), lambda i: (0, i)),
    ],
    out_specs=pl.BlockSpec((NBLK,), lambda i: (i,)),
    out_shape=jax.ShapeDtypeStruct((V,), jnp.uint32),
)


@functools.partial(
    pl.kernel,
    out_type=jax.ShapeDtypeStruct((C, B), jnp.float32),
    mesh=_mesh,
    compiler_params=pltpu.CompilerParams(use_tc_tiling_on_sc=False,
                                         needs_layout_passes=False),
    scratch_types=(
        [pltpu.VMEM((NCHUNK, CHUNK), jnp.int32)]
        + [pltpu.VMEM((CHUNK,), jnp.uint32)
           for _ in range(RING)]
        + [pltpu.VMEM((C, BPW), jnp.float32),
           pltpu.SemaphoreType.DMA]
    ),
)
def _sc_pool(pku, xr, out, idx_v, *rest):
    gbufs, av, sem = rest[:RING], rest[RING], rest[RING + 1]
    c = lax.axis_index("c")
    s = lax.axis_index("s")

    pltpu.sync_copy(xr.at[c, s], idx_v)

    def start(k, r):
        pltpu.async_copy(pku.at[idx_v.at[k]], gbufs[r], sem)

    def wait(k, r):
        pltpu.make_async_copy(pku.at[idx_v.at[k]], gbufs[r], sem).wait()

    for k in range(RING):
        start(k, k)

    def grp_body(ko, carry):
        acc = list(carry)
        for r in range(RING):
            k = ko * RING + r
            wait(k, r)
            for t in range(TPC):
                for g in range(GPB):
                    v = gbufs[r][pl.ds(t * BPW + g * L, L)]
                    a0, a1 = plsc.unpack(
                        plsc.bitcast(v, jnp.bfloat16),
                        format=plsc.PackFormat.INTERLEAVED)
                    acc[g] = acc[g] + a0
                    acc[GPB + g] = acc[GPB + g] + a1
            @pl.when(ko + 1 < NCHUNK // RING)
            def _():
                start(k + RING, r)
        return tuple(acc)

    zero = jnp.zeros((L,), jnp.float32)
    acc = lax.fori_loop(0, NCHUNK // RING, grp_body, (zero,) * (C * GPB))

    for cls in range(C):
        for g in range(GPB):
            av[cls, pl.ds(g * L, L)] = acc[cls * GPB + g]
    pltpu.sync_copy(av, out.at[:, pl.ds((c * NS + s) * BPW, BPW)])


def kernel(x, embed_table, fc_w, fc_b):
    tt = embed_table.T
    w8 = jnp.zeros((8, E), jnp.float32).at[:C].set(fc_w * (1.0 / S))
    b8 = jnp.zeros((8, 128), jnp.float32).at[:C, 0].set(fc_b * (1.0 / S))
    pku = _tc_project(w8, b8, tt)
    xr = (x.reshape(NC, NS, BPW, S).astype(jnp.int32)
          .transpose(0, 1, 3, 2).reshape(NC, NS, NCHUNK, CHUNK))
    out = _sc_pool(pku, xr)
    return out.T


# speedup vs baseline: 5.6650x; 1.0804x over previous
import functools

import jax
import jax.numpy as jnp
import numpy as np
from jax import lax
from jax.experimental import pallas as pl
from jax.experimental.pallas import tpu as pltpu
from jax.experimental.pallas import tpu_sc as plsc

NC, NS, L = 2, 16, 16
NW = NC * NS
V, B, S, E, C = 1000000, 4096, 200, 64, 2
BPW = B // NW
GPB = BPW // L
TPC = 8
CHUNK = TPC * BPW
RING = 5
NCHUNK = S // TPC
NBLK = 32768
GRID = -(-V // NBLK)

_mesh = plsc.VectorSubcoreMesh(core_axis_name="c", subcore_axis_name="s")


def _project_body(w_ref, b_ref, tt_ref, o_ref):
    m = jnp.dot(w_ref[...], tt_ref[...], preferred_element_type=jnp.float32)
    m = m + b_ref[...][:, 0:1]
    u0 = lax.bitcast_convert_type(m[0].astype(jnp.bfloat16), jnp.uint16)
    u1 = lax.bitcast_convert_type(m[1].astype(jnp.bfloat16), jnp.uint16)
    o_ref[...] = (u0.astype(jnp.uint32)
                  | (u1.astype(jnp.uint32) << jnp.uint32(16)))


_tc_project = pl.pallas_call(
    _project_body,
    grid=(GRID,),
    in_specs=[
        pl.BlockSpec((8, E), lambda i: (0, 0)),
        pl.BlockSpec((8, 128), lambda i: (0, 0)),
        pl.BlockSpec((E, NBLK), lambda i: (0, i)),
    ],
    out_specs=pl.BlockSpec((NBLK,), lambda i: (i,)),
    out_shape=jax.ShapeDtypeStruct((V,), jnp.uint32),
)


@functools.partial(
    pl.kernel,
    out_type=jax.ShapeDtypeStruct((C, B), jnp.float32),
    mesh=_mesh,
    compiler_params=pltpu.CompilerParams(use_tc_tiling_on_sc=False,
                                         needs_layout_passes=False),
    scratch_types=(
        [pltpu.VMEM((NCHUNK, CHUNK), jnp.int32)]
        + [pltpu.VMEM((CHUNK,), jnp.uint32)
           for _ in range(RING)]
        + [pltpu.VMEM((C, BPW), jnp.float32),
           pltpu.SemaphoreType.DMA]
    ),
)
def _sc_pool(pku, xr, out, idx_v, *rest):
    gbufs, av, sem = rest[:RING], rest[RING], rest[RING + 1]
    c = lax.axis_index("c")
    s = lax.axis_index("s")

    pltpu.sync_copy(xr.at[c, s], idx_v)

    def start(k, r):
        pltpu.async_copy(pku.at[idx_v.at[k]], gbufs[r], sem)

    def wait(k, r):
        pltpu.make_async_copy(pku.at[idx_v.at[k]], gbufs[r], sem).wait()

    for k in range(RING):
        start(k, k)

    def grp_body(ko, carry):
        acc = list(carry)
        for r in range(RING):
            k = ko * RING + r
            wait(k, r)
            for t in range(TPC):
                for g in range(GPB):
                    v = gbufs[r][pl.ds(t * BPW + g * L, L)]
                    a0, a1 = plsc.unpack(
                        plsc.bitcast(v, jnp.bfloat16),
                        format=plsc.PackFormat.INTERLEAVED)
                    acc[g] = acc[g] + a0
                    acc[GPB + g] = acc[GPB + g] + a1
            @pl.when(ko + 1 < NCHUNK // RING)
            def _():
                start(k + RING, r)
        return tuple(acc)

    zero = jnp.zeros((L,), jnp.float32)
    acc = lax.fori_loop(0, NCHUNK // RING, grp_body, (zero,) * (C * GPB))

    for cls in range(C):
        for g in range(GPB):
            av[cls, pl.ds(g * L, L)] = acc[cls * GPB + g]
    pltpu.sync_copy(av, out.at[:, pl.ds((c * NS + s) * BPW, BPW)])


def kernel(x, embed_table, fc_w, fc_b):
    tt = embed_table.T
    w8 = jnp.zeros((8, E), jnp.float32).at[:C].set(fc_w * (1.0 / S))
    b8 = jnp.zeros((8, 128), jnp.float32).at[:C, 0].set(fc_b * (1.0 / S))
    pku = _tc_project(w8, b8, tt)
    xr = (x.reshape(NC, NS, BPW, S).astype(jnp.int32)
          .transpose(0, 1, 3, 2).reshape(NC, NS, NCHUNK, CHUNK))
    out = _sc_pool(pku, xr)
    return out.T


# speedup vs baseline: 5.7025x; 1.0066x over previous
import functools

import jax
import jax.numpy as jnp
import numpy as np
from jax import lax
from jax.experimental import pallas as pl
from jax.experimental.pallas import tpu as pltpu
from jax.experimental.pallas import tpu_sc as plsc

NC, NS, L = 2, 16, 16
NW = NC * NS
V, B, S, E, C = 1000000, 4096, 200, 64, 2
BPW = B // NW
GPB = BPW // L
TPC = 20
CHUNK = TPC * BPW
RING = 5
NCHUNK = S // TPC
NBLK = 65536
GRID = -(-V // NBLK)

_mesh = plsc.VectorSubcoreMesh(core_axis_name="c", subcore_axis_name="s")


def _project_body(w_ref, b_ref, tt_ref, o_ref):
    m = jnp.dot(w_ref[...], tt_ref[...], preferred_element_type=jnp.float32)
    m = m + b_ref[...][:, 0:1]
    u0 = lax.bitcast_convert_type(m[0].astype(jnp.bfloat16), jnp.uint16)
    u1 = lax.bitcast_convert_type(m[1].astype(jnp.bfloat16), jnp.uint16)
    o_ref[...] = (u0.astype(jnp.uint32)
                  | (u1.astype(jnp.uint32) << jnp.uint32(16)))


_tc_project = pl.pallas_call(
    _project_body,
    grid=(GRID,),
    in_specs=[
        pl.BlockSpec((8, E), lambda i: (0, 0)),
        pl.BlockSpec((8, 128), lambda i: (0, 0)),
        pl.BlockSpec((E, NBLK), lambda i: (0, i)),
    ],
    out_specs=pl.BlockSpec((NBLK,), lambda i: (i,)),
    out_shape=jax.ShapeDtypeStruct((V,), jnp.uint32),
)


@functools.partial(
    pl.kernel,
    out_type=jax.ShapeDtypeStruct((C, B), jnp.float32),
    mesh=_mesh,
    compiler_params=pltpu.CompilerParams(use_tc_tiling_on_sc=False,
                                         needs_layout_passes=False),
    scratch_types=(
        [pltpu.VMEM((NCHUNK, CHUNK), jnp.int32)]
        + [pltpu.VMEM((CHUNK,), jnp.uint32)
           for _ in range(RING)]
        + [pltpu.VMEM((C, BPW), jnp.float32),
           pltpu.SemaphoreType.DMA]
    ),
)
def _sc_pool(pku, xr, out, idx_v, *rest):
    gbufs, av, sem = rest[:RING], rest[RING], rest[RING + 1]
    c = lax.axis_index("c")
    s = lax.axis_index("s")

    pltpu.sync_copy(xr.at[c, s], idx_v)

    def start(k, r):
        pltpu.async_copy(pku.at[idx_v.at[k]], gbufs[r], sem)

    def wait(k, r):
        pltpu.make_async_copy(pku.at[idx_v.at[k]], gbufs[r], sem).wait()

    for k in range(RING):
        start(k, k)

    def grp_body(ko, carry):
        acc = list(carry)
        for r in range(RING):
            k = ko * RING + r
            wait(k, r)
            for t in range(TPC):
                for g in range(GPB):
                    v = gbufs[r][pl.ds(t * BPW + g * L, L)]
                    a0, a1 = plsc.unpack(
                        plsc.bitcast(v, jnp.bfloat16),
                        format=plsc.PackFormat.INTERLEAVED)
                    acc[g] = acc[g] + a0
                    acc[GPB + g] = acc[GPB + g] + a1
            @pl.when(ko + 1 < NCHUNK // RING)
            def _():
                start(k + RING, r)
        return tuple(acc)

    zero = jnp.zeros((L,), jnp.float32)
    acc = lax.fori_loop(0, NCHUNK // RING, grp_body, (zero,) * (C * GPB))

    for cls in range(C):
        for g in range(GPB):
            av[cls, pl.ds(g * L, L)] = acc[cls * GPB + g]
    pltpu.sync_copy(av, out.at[:, pl.ds((c * NS + s) * BPW, BPW)])


def kernel(x, embed_table, fc_w, fc_b):
    tt = embed_table.T
    w8 = jnp.zeros((8, E), jnp.float32).at[:C].set(fc_w * (1.0 / S))
    b8 = jnp.zeros((8, 128), jnp.float32).at[:C, 0].set(fc_b * (1.0 / S))
    pku = _tc_project(w8, b8, tt)
    xr = (x.reshape(NC, NS, BPW, S).astype(jnp.int32)
          .transpose(0, 1, 3, 2).reshape(NC, NS, NCHUNK, CHUNK))
    out = _sc_pool(pku, xr)
    return out.T


# speedup vs baseline: 6.0057x; 1.0532x over previous
import functools

import jax
import jax.numpy as jnp
import numpy as np
from jax import lax
from jax.experimental import pallas as pl
from jax.experimental.pallas import tpu as pltpu
from jax.experimental.pallas import tpu_sc as plsc

NC, NS, L = 2, 16, 16
NW = NC * NS
V, B, S, E, C = 1000000, 4096, 200, 64, 2
BPW = B // NW
GPB = BPW // L
TPC = 20
CHUNK = TPC * BPW
RING = 5
NCHUNK = S // TPC
NBLK = 65536
GRID = -(-V // NBLK)

_mesh = plsc.VectorSubcoreMesh(core_axis_name="c", subcore_axis_name="s")


def _project_body(w_ref, b_ref, tt_ref, o_ref):
    m = jnp.dot(w_ref[...], tt_ref[...], preferred_element_type=jnp.float32)
    m = m + b_ref[...][:, 0:1]
    u0 = lax.bitcast_convert_type(m[0].astype(jnp.bfloat16), jnp.uint16)
    u1 = lax.bitcast_convert_type(m[1].astype(jnp.bfloat16), jnp.uint16)
    o_ref[...] = (u0.astype(jnp.uint32)
                  | (u1.astype(jnp.uint32) << jnp.uint32(16)))


_tc_project = pl.pallas_call(
    _project_body,
    grid=(GRID,),
    in_specs=[
        pl.BlockSpec((8, E), lambda i: (0, 0)),
        pl.BlockSpec((8, 128), lambda i: (0, 0)),
        pl.BlockSpec((E, NBLK), lambda i: (0, i)),
    ],
    out_specs=pl.BlockSpec((NBLK,), lambda i: (i,)),
    out_shape=jax.ShapeDtypeStruct((V,), jnp.uint32),
)


@functools.partial(
    pl.kernel,
    out_type=jax.ShapeDtypeStruct((C, B), jnp.float32),
    mesh=_mesh,
    compiler_params=pltpu.CompilerParams(use_tc_tiling_on_sc=False,
                                         needs_layout_passes=False),
    scratch_types=(
        [pltpu.VMEM((NCHUNK, CHUNK), jnp.int32)]
        + [pltpu.VMEM((CHUNK,), jnp.uint32)
           for _ in range(NCHUNK)]
        + [pltpu.VMEM((C, BPW), jnp.float32),
           pltpu.VMEM_SHARED((V,), jnp.uint32),
           pltpu.SemaphoreType.DMA,
           pltpu.SemaphoreType.DMA]
    ),
)
def _sc_pool(pku, xr, out, idx_v, *rest):
    gbufs = rest[:NCHUNK]
    av, ptab = rest[NCHUNK], rest[NCHUNK + 1]
    sems = {0: rest[NCHUNK + 2], 1: rest[NCHUNK + 3]}
    c = lax.axis_index("c")
    s = lax.axis_index("s")

    pltpu.sync_copy(xr.at[c, s], idx_v)
    @pl.when(s < 8)
    def _():
        pltpu.sync_copy(pku.at[pl.ds(s * (V // 8), V // 8)],
                        ptab.at[pl.ds(s * (V // 8), V // 8)])
    plsc.subcore_barrier()

    def start(k, r, src, q):
        pltpu.async_copy(src.at[idx_v.at[k]], gbufs[r], sems[q])

    def wait(k, r, src, q):
        pltpu.make_async_copy(src.at[idx_v.at[k]], gbufs[r], sems[q]).wait()

    acc = [jnp.zeros((L,), jnp.float32) for _ in range(C * GPB)]

    def accumulate(r):
        for t in range(TPC):
            for g in range(GPB):
                v = gbufs[r][pl.ds(t * BPW + g * L, L)]
                a0, a1 = plsc.unpack(
                    plsc.bitcast(v, jnp.bfloat16),
                    format=plsc.PackFormat.INTERLEAVED)
                acc[g] = acc[g] + a0
                acc[GPB + g] = acc[GPB + g] + a1

    srcs = [pku if k < 4 else ptab for k in range(NCHUNK)]
    qs = [0 if k < 4 else 1 for k in range(NCHUNK)]
    for k in range(NCHUNK):
        start(k, k, srcs[k], qs[k])
    for k in range(NCHUNK):
        wait(k, k, srcs[k], qs[k])
        accumulate(k)

    for cls in range(C):
        for g in range(GPB):
            av[cls, pl.ds(g * L, L)] = acc[cls * GPB + g]
    pltpu.sync_copy(av, out.at[:, pl.ds((c * NS + s) * BPW, BPW)])


def kernel(x, embed_table, fc_w, fc_b):
    tt = embed_table.T
    w8 = jnp.zeros((8, E), jnp.float32).at[:C].set(fc_w * (1.0 / S))
    b8 = jnp.zeros((8, 128), jnp.float32).at[:C, 0].set(fc_b * (1.0 / S))
    pku = _tc_project(w8, b8, tt)
    xr = (x.reshape(NC, NS, BPW, S).astype(jnp.int32)
          .transpose(0, 1, 3, 2).reshape(NC, NS, NCHUNK, CHUNK))
    out = _sc_pool(pku, xr)
    return out.T


# speedup vs baseline: 6.1434x; 1.0229x over previous
import functools

import jax
import jax.numpy as jnp
import numpy as np
from jax import lax
from jax.experimental import pallas as pl
from jax.experimental.pallas import tpu as pltpu
from jax.experimental.pallas import tpu_sc as plsc

NC, NS, L = 2, 16, 16
NW = NC * NS
V, B, S, E, C = 1000000, 4096, 200, 64, 2
BPW = B // NW
GPB = BPW // L
TPC = 20
CHUNK = TPC * BPW
RING = 5
NCHUNK = S // TPC
NBLK = 65536
GRID = -(-V // NBLK)

_mesh = plsc.VectorSubcoreMesh(core_axis_name="c", subcore_axis_name="s")


def _project_body(w_ref, b_ref, tt_ref, o_ref):
    m = jnp.dot(w_ref[...], tt_ref[...], preferred_element_type=jnp.float32)
    m = m + b_ref[...][:, 0:1]
    u0 = lax.bitcast_convert_type(m[0].astype(jnp.bfloat16), jnp.uint16)
    u1 = lax.bitcast_convert_type(m[1].astype(jnp.bfloat16), jnp.uint16)
    o_ref[...] = (u0.astype(jnp.uint32)
                  | (u1.astype(jnp.uint32) << jnp.uint32(16)))


_tc_project = pl.pallas_call(
    _project_body,
    grid=(GRID,),
    in_specs=[
        pl.BlockSpec((8, E), lambda i: (0, 0)),
        pl.BlockSpec((8, 128), lambda i: (0, 0)),
        pl.BlockSpec((E, NBLK), lambda i: (0, i)),
    ],
    out_specs=pl.BlockSpec((NBLK,), lambda i: (i,)),
    out_shape=jax.ShapeDtypeStruct((V,), jnp.uint32),
)


@functools.partial(
    pl.kernel,
    out_type=jax.ShapeDtypeStruct((C, B), jnp.float32),
    mesh=_mesh,
    compiler_params=pltpu.CompilerParams(use_tc_tiling_on_sc=False,
                                         needs_layout_passes=False),
    scratch_types=(
        [pltpu.VMEM((NCHUNK, CHUNK), jnp.int32)]
        + [pltpu.VMEM((CHUNK,), jnp.uint32)
           for _ in range(NCHUNK)]
        + [pltpu.VMEM((C, BPW), jnp.float32),
           pltpu.VMEM_SHARED((V,), jnp.uint32),
           pltpu.SemaphoreType.DMA,
           pltpu.SemaphoreType.DMA]
    ),
)
def _sc_pool(pku, xr, out, idx_v, *rest):
    gbufs = rest[:NCHUNK]
    av, ptab = rest[NCHUNK], rest[NCHUNK + 1]
    sems = {0: rest[NCHUNK + 2], 1: rest[NCHUNK + 3]}
    c = lax.axis_index("c")
    s = lax.axis_index("s")

    pltpu.sync_copy(xr.at[c, s], idx_v)
    @pl.when(s < 8)
    def _():
        pltpu.sync_copy(pku.at[pl.ds(s * (V // 8), V // 8)],
                        ptab.at[pl.ds(s * (V // 8), V // 8)])
    plsc.subcore_barrier()

    def start(k, r, src, q):
        pltpu.async_copy(src.at[idx_v.at[k]], gbufs[r], sems[q])

    def wait(k, r, src, q):
        pltpu.make_async_copy(src.at[idx_v.at[k]], gbufs[r], sems[q]).wait()

    acc = [jnp.zeros((L,), jnp.float32) for _ in range(C * GPB)]

    def accumulate(r):
        for t in range(TPC):
            for g in range(GPB):
                v = gbufs[r][pl.ds(t * BPW + g * L, L)]
                a0, a1 = plsc.unpack(
                    plsc.bitcast(v, jnp.bfloat16),
                    format=plsc.PackFormat.INTERLEAVED)
                acc[g] = acc[g] + a0
                acc[GPB + g] = acc[GPB + g] + a1

    srcs = [pku if k < 3 else ptab for k in range(NCHUNK)]
    qs = [0 if k < 3 else 1 for k in range(NCHUNK)]
    for k in range(NCHUNK):
        start(k, k, srcs[k], qs[k])
    for k in range(NCHUNK):
        wait(k, k, srcs[k], qs[k])
        accumulate(k)

    for cls in range(C):
        for g in range(GPB):
            av[cls, pl.ds(g * L, L)] = acc[cls * GPB + g]
    pltpu.sync_copy(av, out.at[:, pl.ds((c * NS + s) * BPW, BPW)])


def kernel(x, embed_table, fc_w, fc_b):
    tt = embed_table.T
    w8 = jnp.zeros((8, E), jnp.float32).at[:C].set(fc_w * (1.0 / S))
    b8 = jnp.zeros((8, 128), jnp.float32).at[:C, 0].set(fc_b * (1.0 / S))
    pku = _tc_project(w8, b8, tt)
    xr = (x.reshape(NC, NS, BPW, S).astype(jnp.int32)
          .transpose(0, 1, 3, 2).reshape(NC, NS, NCHUNK, CHUNK))
    out = _sc_pool(pku, xr)
    return out.T


# speedup vs baseline: 6.3911x; 1.0403x over previous
import functools

import jax
import jax.numpy as jnp
import numpy as np
from jax import lax
from jax.experimental import pallas as pl
from jax.experimental.pallas import tpu as pltpu
from jax.experimental.pallas import tpu_sc as plsc

NC, NS, L = 2, 16, 16
NW = NC * NS
V, B, S, E, C = 1000000, 4096, 200, 64, 2
BPW = B // NW
GPB = BPW // L
TPC = 20
CHUNK = TPC * BPW
RING = 5
NCHUNK = S // TPC
NBLK = 65536
GRID = -(-V // NBLK)

_mesh = plsc.VectorSubcoreMesh(core_axis_name="c", subcore_axis_name="s")


def _project_body(w_ref, b_ref, tt_ref, o_ref):
    m = jnp.dot(w_ref[...], tt_ref[...], preferred_element_type=jnp.float32)
    m = m + b_ref[...][:, 0:1]
    u0 = lax.bitcast_convert_type(m[0].astype(jnp.bfloat16), jnp.uint16)
    u1 = lax.bitcast_convert_type(m[1].astype(jnp.bfloat16), jnp.uint16)
    o_ref[...] = (u0.astype(jnp.uint32)
                  | (u1.astype(jnp.uint32) << jnp.uint32(16)))


_tc_project = pl.pallas_call(
    _project_body,
    grid=(GRID,),
    in_specs=[
        pl.BlockSpec((8, E), lambda i: (0, 0)),
        pl.BlockSpec((8, 128), lambda i: (0, 0)),
        pl.BlockSpec((E, NBLK), lambda i: (0, i)),
    ],
    out_specs=pl.BlockSpec((NBLK,), lambda i: (i,)),
    out_shape=jax.ShapeDtypeStruct((V,), jnp.uint32),
)


@functools.partial(
    pl.kernel,
    out_type=jax.ShapeDtypeStruct((C, B), jnp.float32),
    mesh=_mesh,
    compiler_params=pltpu.CompilerParams(use_tc_tiling_on_sc=False,
                                         needs_layout_passes=False),
    scratch_types=(
        [pltpu.VMEM((NCHUNK, CHUNK), jnp.int32)]
        + [pltpu.VMEM((CHUNK,), jnp.uint32)
           for _ in range(NCHUNK)]
        + [pltpu.VMEM((C, BPW), jnp.float32),
           pltpu.VMEM_SHARED((V,), jnp.uint32),
           pltpu.SemaphoreType.DMA,
           pltpu.SemaphoreType.DMA]
    ),
)
def _sc_pool(pku, xr, out, idx_v, *rest):
    gbufs = rest[:NCHUNK]
    av, ptab = rest[NCHUNK], rest[NCHUNK + 1]
    sems = {0: rest[NCHUNK + 2], 1: rest[NCHUNK + 3]}
    c = lax.axis_index("c")
    s = lax.axis_index("s")

    pltpu.sync_copy(xr.at[c, s], idx_v)
    @pl.when(s < 8)
    def _():
        pltpu.sync_copy(pku.at[pl.ds(s * (V // 8), V // 8)],
                        ptab.at[pl.ds(s * (V // 8), V // 8)])
    plsc.subcore_barrier()

    def start(k, r, src, q):
        pltpu.async_copy(src.at[idx_v.at[k]], gbufs[r], sems[q])

    def wait(k, r, src, q):
        pltpu.make_async_copy(src.at[idx_v.at[k]], gbufs[r], sems[q]).wait()

    acc = [jnp.zeros((L,), jnp.float32) for _ in range(C * GPB)]

    def accumulate(r):
        for t in range(TPC):
            for g in range(GPB):
                v = gbufs[r][pl.ds(t * BPW + g * L, L)]
                a0, a1 = plsc.unpack(
                    plsc.bitcast(v, jnp.bfloat16),
                    format=plsc.PackFormat.INTERLEAVED)
                acc[g] = acc[g] + a0
                acc[GPB + g] = acc[GPB + g] + a1

    srcs = [pku if k < 2 else ptab for k in range(NCHUNK)]
    qs = [0 if k < 2 else 1 for k in range(NCHUNK)]
    for k in range(NCHUNK):
        start(k, k, srcs[k], qs[k])
    for k in range(NCHUNK):
        wait(k, k, srcs[k], qs[k])
        accumulate(k)

    for cls in range(C):
        for g in range(GPB):
            av[cls, pl.ds(g * L, L)] = acc[cls * GPB + g]
    pltpu.sync_copy(av, out.at[:, pl.ds((c * NS + s) * BPW, BPW)])


def kernel(x, embed_table, fc_w, fc_b):
    tt = embed_table.T
    w8 = jnp.zeros((8, E), jnp.float32).at[:C].set(fc_w * (1.0 / S))
    b8 = jnp.zeros((8, 128), jnp.float32).at[:C, 0].set(fc_b * (1.0 / S))
    pku = _tc_project(w8, b8, tt)
    xr = (x.reshape(NC, NS, BPW, S).astype(jnp.int32)
          .transpose(0, 1, 3, 2).reshape(NC, NS, NCHUNK, CHUNK))
    out = _sc_pool(pku, xr)
    return out.T


# speedup vs baseline: 6.4310x; 1.0062x over previous
import functools

import jax
import jax.numpy as jnp
import numpy as np
from jax import lax
from jax.experimental import pallas as pl
from jax.experimental.pallas import tpu as pltpu
from jax.experimental.pallas import tpu_sc as plsc

NC, NS, L = 2, 16, 16
NW = NC * NS
V, B, S, E, C = 1000000, 4096, 200, 64, 2
BPW = B // NW
GPB = BPW // L
TPC = 20
CHUNK = TPC * BPW
RING = 5
NCHUNK = S // TPC
NBLK = 65536
GRID = -(-V // NBLK)

_mesh = plsc.VectorSubcoreMesh(core_axis_name="c", subcore_axis_name="s")


def _project_body(w_ref, b_ref, tt_ref, o_ref):
    m = jnp.dot(w_ref[...], tt_ref[...], preferred_element_type=jnp.float32)
    m = m + b_ref[...][:, 0:1]
    u0 = lax.bitcast_convert_type(m[0].astype(jnp.bfloat16), jnp.uint16)
    u1 = lax.bitcast_convert_type(m[1].astype(jnp.bfloat16), jnp.uint16)
    o_ref[...] = (u0.astype(jnp.uint32)
                  | (u1.astype(jnp.uint32) << jnp.uint32(16)))


_tc_project = pl.pallas_call(
    _project_body,
    grid=(GRID,),
    in_specs=[
        pl.BlockSpec((8, E), lambda i: (0, 0)),
        pl.BlockSpec((8, 128), lambda i: (0, 0)),
        pl.BlockSpec((E, NBLK), lambda i: (0, i)),
    ],
    out_specs=pl.BlockSpec((NBLK,), lambda i: (i,)),
    out_shape=jax.ShapeDtypeStruct((V,), jnp.uint32),
)


@functools.partial(
    pl.kernel,
    out_type=jax.ShapeDtypeStruct((C, B), jnp.float32),
    mesh=_mesh,
    compiler_params=pltpu.CompilerParams(use_tc_tiling_on_sc=False,
                                         needs_layout_passes=False),
    scratch_types=(
        [pltpu.VMEM((NCHUNK, CHUNK), jnp.int32)]
        + [pltpu.VMEM((CHUNK,), jnp.uint32)
           for _ in range(NCHUNK)]
        + [pltpu.VMEM((C, BPW), jnp.float32),
           pltpu.VMEM_SHARED((V,), jnp.uint32),
           pltpu.SemaphoreType.DMA,
           pltpu.SemaphoreType.DMA]
    ),
)
def _sc_pool(pku, xr, out, idx_v, *rest):
    gbufs = rest[:NCHUNK]
    av, ptab = rest[NCHUNK], rest[NCHUNK + 1]
    sems = {0: rest[NCHUNK + 2], 1: rest[NCHUNK + 3]}
    c = lax.axis_index("c")
    s = lax.axis_index("s")

    pltpu.sync_copy(xr.at[c, s], idx_v)
    @pl.when(s < 8)
    def _():
        pltpu.sync_copy(pku.at[pl.ds(s * (V // 8), V // 8)],
                        ptab.at[pl.ds(s * (V // 8), V // 8)])
    plsc.subcore_barrier()

    def start(k, r, src, q):
        pltpu.async_copy(src.at[idx_v.at[k]], gbufs[r], sems[q])

    def wait(k, r, src, q):
        pltpu.make_async_copy(src.at[idx_v.at[k]], gbufs[r], sems[q]).wait()

    acc = [jnp.zeros((L,), jnp.float32) for _ in range(C * GPB)]

    def accumulate(r):
        for t in range(TPC):
            for g in range(GPB):
                v = gbufs[r][pl.ds(t * BPW + g * L, L)]
                a0, a1 = plsc.unpack(
                    plsc.bitcast(v, jnp.bfloat16),
                    format=plsc.PackFormat.INTERLEAVED)
                acc[g] = acc[g] + a0
                acc[GPB + g] = acc[GPB + g] + a1

    srcs = [pku if k < 1 else ptab for k in range(NCHUNK)]
    qs = [0 if k < 1 else 1 for k in range(NCHUNK)]
    for k in range(NCHUNK):
        start(k, k, srcs[k], qs[k])
    for k in range(NCHUNK):
        wait(k, k, srcs[k], qs[k])
        accumulate(k)

    for cls in range(C):
        for g in range(GPB):
            av[cls, pl.ds(g * L, L)] = acc[cls * GPB + g]
    pltpu.sync_copy(av, out.at[:, pl.ds((c * NS + s) * BPW, BPW)])


def kernel(x, embed_table, fc_w, fc_b):
    tt = embed_table.T
    w8 = jnp.zeros((8, E), jnp.float32).at[:C].set(fc_w * (1.0 / S))
    b8 = jnp.zeros((8, 128), jnp.float32).at[:C, 0].set(fc_b * (1.0 / S))
    pku = _tc_project(w8, b8, tt)
    xr = (x.reshape(NC, NS, BPW, S).astype(jnp.int32)
          .transpose(0, 1, 3, 2).reshape(NC, NS, NCHUNK, CHUNK))
    out = _sc_pool(pku, xr)
    return out.T
